# Initial kernel scaffold; baseline (speedup 1.0000x reference)
import jax
import jax.numpy as jnp
from jax.experimental import pallas as pl


def kernel(x, gate_w, w_gate, w_up, w_down):
    raise NotImplementedError("write your pallas kernel here")


# speedup vs baseline: 1.3244x; 1.3244x over previous
import jax
import jax.numpy as jnp
from jax.experimental import pallas as pl
from jax.experimental.pallas import tpu as pltpu

HIDDEN = 1024
FFN = 4096
E = 8
TOKENS = 2048

FT = 512
NF = FFN // FT
TCH = 256
NTCH = TOKENS // TCH


def _router_body(x_ref, gw_ref, logits_ref, we_ref):
    xf = x_ref[...]
    gw = gw_ref[...]
    logits = jnp.dot(xf, gw, preferred_element_type=jnp.float32)
    logits_ref[...] = logits
    rw = jax.nn.softmax(logits, axis=-1)
    ids = jax.lax.broadcasted_iota(jnp.int32, rw.shape, 1)
    m1 = jnp.max(rw, axis=-1, keepdims=True)
    e1 = jnp.min(jnp.where(rw == m1, ids, E), axis=-1, keepdims=True)
    rw2 = jnp.where(ids == e1, -jnp.inf, rw)
    m2 = jnp.max(rw2, axis=-1, keepdims=True)
    e2 = jnp.min(jnp.where(rw2 == m2, ids, E), axis=-1, keepdims=True)
    s = m1 + m2
    w1 = m1 / s
    w2 = m2 / s
    we_ref[...] = jnp.where(ids == e1, w1, 0.0) + jnp.where(ids == e2, w2, 0.0)


def _ffn_body(x_ref, we_ref, wg_ref, wu_ref, wd_ref, out_ref):
    e = pl.program_id(0)
    f = pl.program_id(1)
    wg = wg_ref[0].astype(jnp.bfloat16)
    wu = wu_ref[0].astype(jnp.bfloat16)
    wd = wd_ref[0].astype(jnp.bfloat16)

    @pl.when((e == 0) & (f == 0))
    def _init():
        out_ref[...] = jnp.zeros_like(out_ref)

    ids = jax.lax.broadcasted_iota(jnp.int32, (TCH, E), 1)
    for t in range(NTCH):
        xt = x_ref[pl.ds(t * TCH, TCH), :].astype(jnp.bfloat16)
        g = jnp.dot(xt, wg, preferred_element_type=jnp.float32)
        u = jnp.dot(xt, wu, preferred_element_type=jnp.float32)
        h = (g * jax.nn.sigmoid(g)) * u
        we_chunk = we_ref[pl.ds(t * TCH, TCH), :]
        w_col = jnp.sum(jnp.where(ids == e, we_chunk, 0.0),
                        axis=1, keepdims=True)
        hb = (h * w_col).astype(jnp.bfloat16)
        out_ref[pl.ds(t * TCH, TCH), :] += jnp.dot(
            hb, wd, preferred_element_type=jnp.float32)


def kernel(x, gate_w, w_gate, w_up, w_down):
    B, S, D = x.shape
    xf = x.reshape(S, D)

    logits, we = pl.pallas_call(
        _router_body,
        out_shape=(
            jax.ShapeDtypeStruct((TOKENS, E), jnp.float32),
            jax.ShapeDtypeStruct((TOKENS, E), jnp.float32),
        ),
    )(xf, gate_w)

    out = pl.pallas_call(
        _ffn_body,
        grid=(E, NF),
        in_specs=[
            pl.BlockSpec((TOKENS, HIDDEN), lambda e, f: (0, 0)),
            pl.BlockSpec((TOKENS, E), lambda e, f: (0, 0)),
            pl.BlockSpec((1, HIDDEN, FT), lambda e, f: (e, 0, f)),
            pl.BlockSpec((1, HIDDEN, FT), lambda e, f: (e, 0, f)),
            pl.BlockSpec((1, FT, HIDDEN), lambda e, f: (e, f, 0)),
        ],
        out_specs=pl.BlockSpec((TOKENS, HIDDEN), lambda e, f: (0, 0)),
        out_shape=jax.ShapeDtypeStruct((TOKENS, HIDDEN), jnp.float32),
    )(xf, we, w_gate, w_up, w_down)

    return out.reshape(B, S, D), logits
